# Initial kernel scaffold; baseline (speedup 1.0000x reference)
import jax
import jax.numpy as jnp
from jax.experimental import pallas as pl


def kernel(x, edge_index, W1, b1, W2, b2):
    raise NotImplementedError("write your pallas kernel here")


# speedup vs baseline: 24.3147x; 24.3147x over previous
import functools

import jax
import jax.numpy as jnp
from jax import lax
from jax.experimental import pallas as pl
from jax.experimental.pallas import tpu as pltpu
from jax.experimental.pallas import tpu_sc as plsc

N = 10000
NP = 10240
F = 128
E = 320000

NC = 2
NS = 16
NW = NC * NS
L = 16

CHUNK = 128
KCH = 79
EPW = CHUNK * KCH
EPAD = NW * EPW

_mesh = plsc.VectorSubcoreMesh(core_axis_name="c", subcore_axis_name="s")
_sc_params = pltpu.CompilerParams(needs_layout_passes=False)


@functools.partial(
    pl.kernel,
    out_type=jax.ShapeDtypeStruct((NW, NP), jnp.float32),
    mesh=_mesh,
    compiler_params=_sc_params,
    scratch_types=[
        pltpu.VMEM((EPW,), jnp.int32),
        pltpu.VMEM((NP,), jnp.float32),
    ],
)
def _sc_degree(dst_hbm, out_hbm, dst_v, acc_v):
    wid = lax.axis_index("c") * NS + lax.axis_index("s")
    pltpu.sync_copy(dst_hbm.at[wid], dst_v)

    def zero(i, _):
        acc_v[pl.ds(pl.multiple_of(i * L, L), L)] = jnp.zeros((L,), jnp.float32)
        return _

    lax.fori_loop(0, NP // L, zero, 0)

    ones = jnp.full((L,), 1.0, jnp.float32)

    def body(i, _):
        idx = dst_v[pl.ds(pl.multiple_of(i * L, L), L)]
        plsc.addupdate_scatter(acc_v, [idx], ones)
        return _

    lax.fori_loop(0, EPW // L, body, 0)
    pltpu.sync_copy(acc_v, out_hbm.at[wid])


@functools.partial(
    pl.kernel,
    out_type=jax.ShapeDtypeStruct((NC, NP, F), jnp.float32),
    mesh=_mesh,
    compiler_params=_sc_params,
    scratch_types=[
        pltpu.VMEM((KCH, CHUNK), jnp.int32),
        pltpu.VMEM((KCH, CHUNK), jnp.int32),
        pltpu.VMEM((CHUNK, F), jnp.float32),
        pltpu.VMEM_SHARED((NP, F), jnp.float32),
        pltpu.SemaphoreType.DMA,
    ],
)
def _sc_agg_rows(hs_hbm, src_hbm, dst_hbm, zeros_hbm, out_hbm,
                 src_v, dst_v, rows_v, acc_sh, sem):
    c = lax.axis_index("c")
    s = lax.axis_index("s")
    wid = c * NS + s
    pltpu.sync_copy(src_hbm.at[wid], src_v)
    pltpu.sync_copy(dst_hbm.at[wid], dst_v)

    rows_per_tile = NP // NS
    pltpu.sync_copy(zeros_hbm.at[pl.ds(s * rows_per_tile, rows_per_tile)],
                    acc_sh.at[pl.ds(s * rows_per_tile, rows_per_tile)])
    plsc.subcore_barrier()

    def body(j, _):
        pltpu.async_copy(hs_hbm.at[src_v.at[j]], rows_v, sem).wait()
        pltpu.sync_copy(rows_v, acc_sh.at[dst_v.at[j]], add=True)
        return _

    lax.fori_loop(0, KCH, body, 0)
    plsc.subcore_barrier()

    @pl.when(s == 0)
    def _():
        pltpu.sync_copy(acc_sh, out_hbm.at[c])


@functools.partial(
    pl.kernel,
    out_type=jax.ShapeDtypeStruct((NW, NP), jnp.float32),
    mesh=_mesh,
    compiler_params=_sc_params,
    scratch_types=[
        pltpu.VMEM((EPW,), jnp.int32),
        pltpu.VMEM((EPW,), jnp.int32),
        pltpu.VMEM((NP,), jnp.float32),
        pltpu.VMEM((NP,), jnp.float32),
    ],
)
def _sc_agg_scalar(src_hbm, dst_hbm, zs_hbm, out_hbm, src_v, dst_v, zs_v, acc_v):
    wid = lax.axis_index("c") * NS + lax.axis_index("s")
    pltpu.sync_copy(src_hbm.at[wid], src_v)
    pltpu.sync_copy(dst_hbm.at[wid], dst_v)
    pltpu.sync_copy(zs_hbm, zs_v)

    def zero(i, _):
        acc_v[pl.ds(pl.multiple_of(i * L, L), L)] = jnp.zeros((L,), jnp.float32)
        return _

    lax.fori_loop(0, NP // L, zero, 0)

    def body(i, _):
        off = pl.multiple_of(i * L, L)
        sidx = src_v[pl.ds(off, L)]
        didx = dst_v[pl.ds(off, L)]
        vals = plsc.load_gather(zs_v, [sidx])
        plsc.addupdate_scatter(acc_v, [didx], vals)
        return _

    lax.fori_loop(0, EPW // L, body, 0)
    pltpu.sync_copy(acc_v, out_hbm.at[wid])


R = 2048


def _tc1_body(part_ref, x_ref, w1_ref, hs_ref, dis_ref):
    ones = jnp.ones((NW, 1), jnp.float32)
    deg = lax.dot_general(part_ref[...], ones, (((0,), (0,)), ((), ())),
                          preferred_element_type=jnp.float32)
    dis = jnp.where(deg > 0.0, lax.rsqrt(deg), 0.0)
    h = jnp.dot(x_ref[...], w1_ref[...], preferred_element_type=jnp.float32)
    hs_ref[...] = h * dis
    dis_ref[...] = dis


def _tc2_body(a0_ref, a1_ref, hs_ref, dis_ref, b1_ref, w2_ref, zs_ref):
    dis = dis_ref[...]
    h1 = dis * (a0_ref[...] + a1_ref[...] + hs_ref[...]) + b1_ref[...]
    h1 = jnp.maximum(h1, 0.0)
    z = jnp.dot(h1, w2_ref[...], preferred_element_type=jnp.float32)
    zs_ref[...] = z * dis


def _tc3_body(part_ref, zs_ref, dis_ref, b2_ref, out_ref):
    ones = jnp.ones((NW, 1), jnp.float32)
    acc = lax.dot_general(part_ref[...], ones, (((0,), (0,)), ((), ())),
                          preferred_element_type=jnp.float32)
    pre = dis_ref[...] * (acc + zs_ref[...]) + b2_ref[...]
    out_ref[...] = 1.0 / (1.0 + jnp.exp(-pre))


def _tc1(partials, x_pad, W1):
    return pl.pallas_call(
        _tc1_body,
        grid=(NP // R,),
        in_specs=[
            pl.BlockSpec((NW, R), lambda i: (0, i)),
            pl.BlockSpec((R, F), lambda i: (i, 0)),
            pl.BlockSpec((F, F), lambda i: (0, 0)),
        ],
        out_specs=[
            pl.BlockSpec((R, F), lambda i: (i, 0)),
            pl.BlockSpec((R, 1), lambda i: (i, 0)),
        ],
        out_shape=[
            jax.ShapeDtypeStruct((NP, F), jnp.float32),
            jax.ShapeDtypeStruct((NP, 1), jnp.float32),
        ],
    )(partials, x_pad, W1)


def _tc2(a0, a1, hs, dis, b1r, W2):
    return pl.pallas_call(
        _tc2_body,
        grid=(NP // R,),
        in_specs=[
            pl.BlockSpec((R, F), lambda i: (i, 0)),
            pl.BlockSpec((R, F), lambda i: (i, 0)),
            pl.BlockSpec((R, F), lambda i: (i, 0)),
            pl.BlockSpec((R, 1), lambda i: (i, 0)),
            pl.BlockSpec((1, F), lambda i: (0, 0)),
            pl.BlockSpec((F, 1), lambda i: (0, 0)),
        ],
        out_specs=pl.BlockSpec((R, 1), lambda i: (i, 0)),
        out_shape=jax.ShapeDtypeStruct((NP, 1), jnp.float32),
    )(a0, a1, hs, dis, b1r, W2)


def _tc3(partials, zs, dis, b2r):
    return pl.pallas_call(
        _tc3_body,
        grid=(NP // R,),
        in_specs=[
            pl.BlockSpec((NW, R), lambda i: (0, i)),
            pl.BlockSpec((R, 1), lambda i: (i, 0)),
            pl.BlockSpec((R, 1), lambda i: (i, 0)),
            pl.BlockSpec((1, 1), lambda i: (0, 0)),
        ],
        out_specs=pl.BlockSpec((R, 1), lambda i: (i, 0)),
        out_shape=jax.ShapeDtypeStruct((NP, 1), jnp.float32),
    )(partials, zs, dis, b2r)


@jax.jit
def kernel(x, edge_index, W1, b1, W2, b2):
    src = edge_index[0].astype(jnp.int32)
    dst = edge_index[1].astype(jnp.int32)
    pad = jnp.full((EPAD - E,), N, jnp.int32)
    src_p = jnp.concatenate([src, pad]).reshape(NW, EPW)
    dst_p = jnp.concatenate([dst, pad]).reshape(NW, EPW)
    src3 = src_p.reshape(NW, KCH, CHUNK)
    dst3 = dst_p.reshape(NW, KCH, CHUNK)

    x_pad = jnp.zeros((NP, F), x.dtype).at[:N].set(x)
    zeros_rows = jnp.zeros((NP, F), jnp.float32)

    deg_part = _sc_degree(dst_p)
    hs, dis = _tc1(deg_part, x_pad, W1)
    pacc = _sc_agg_rows(hs, src3, dst3, zeros_rows)
    zs = _tc2(pacc[0], pacc[1], hs, dis,
              b1.reshape(1, F), W2)
    pacc2 = _sc_agg_scalar(src_p, dst_p, zs.reshape(NP))
    out = _tc3(pacc2, zs, dis, b2.reshape(1, 1))
    return out[:N]
